# Initial kernel scaffold; baseline (speedup 1.0000x reference)
import jax
import jax.numpy as jnp
from jax.experimental import pallas as pl


def kernel(x, edge_index, edge_mask, W1, b1, W2, b2):
    raise NotImplementedError("write your pallas kernel here")


# speedup vs baseline: 7.0816x; 7.0816x over previous
import functools

import jax
import jax.numpy as jnp
from jax import lax
from jax.experimental import pallas as pl
from jax.experimental.pallas import tpu as pltpu
from jax.experimental.pallas import tpu_sc as plsc

N = 10000
E = 320000
H = 128

NC = 2
NS = 16
NW = NC * NS
EPT = E // NW
K = 80
NCHUNK = EPT // K
NPAD = 10240
RPT = NPAD // NS
ZR = 128
DSL = NPAD // NS

_mesh = plsc.VectorSubcoreMesh(
    core_axis_name="c", subcore_axis_name="s", num_cores=NC, num_subcores=NS)


def _deg_body(col_hbm, w_hbm, out_hbm, col_v, w_v, zbuf, deg_sh, sem):
    c = lax.axis_index("c")
    s = lax.axis_index("s")
    wid = c * NS + s
    zero16 = jnp.zeros((16,), jnp.float32)

    def zb(r, carry):
        zbuf[pl.ds(r * 16, 16)] = zero16
        return carry
    lax.fori_loop(0, DSL // 16, zb, 0)
    pltpu.sync_copy(zbuf, deg_sh.at[pl.ds(s * DSL, DSL)])

    pltpu.sync_copy(col_hbm.at[wid], col_v)
    pltpu.sync_copy(w_hbm.at[wid], w_v)

    plsc.subcore_barrier()

    def chunk(i, carry):
        pltpu.sync_copy(w_v.at[i], deg_sh.at[col_v.at[i]], add=True)
        return carry
    lax.fori_loop(0, NCHUNK, chunk, 0)

    plsc.subcore_barrier()
    pltpu.sync_copy(deg_sh.at[pl.ds(s * DSL, DSL)],
                    out_hbm.at[c].at[pl.ds(s * DSL, DSL)])


_deg_kernel = functools.partial(
    pl.kernel,
    out_type=jax.ShapeDtypeStruct((NC, NPAD), jnp.float32),
    mesh=_mesh,
    scratch_types=[
        pltpu.VMEM((NCHUNK, K), jnp.int32),
        pltpu.VMEM((NCHUNK, K), jnp.float32),
        pltpu.VMEM((DSL,), jnp.float32),
        pltpu.VMEM_SHARED((NPAD,), jnp.float32),
        pltpu.SemaphoreType.DMA,
    ],
)(_deg_body)


def _spmm_body(xs_hbm, row_hbm, col_hbm, wrep_hbm, out_hbm,
               row2, col2, wrep2, rows2, acc_sh, semi, semg):
    c = lax.axis_index("c")
    s = lax.axis_index("s")
    wid = c * NS + s
    zero16 = jnp.zeros((16,), jnp.float32)

    def zb(r, carry):
        for j in range(H // 16):
            rows2[0, r, pl.ds(j * 16, 16)] = zero16
        return carry
    lax.fori_loop(0, K, zb, 0)
    for k in range(RPT // K):
        pltpu.sync_copy(rows2.at[0], acc_sh.at[pl.ds(s * RPT + k * K, K)])

    plsc.subcore_barrier()

    pltpu.sync_copy(row_hbm.at[wid].at[0], row2.at[0])
    pltpu.sync_copy(col_hbm.at[wid].at[0], col2.at[0])
    pltpu.sync_copy(wrep_hbm.at[wid].at[0], wrep2.at[0])
    pltpu.async_copy(xs_hbm.at[row2.at[0]], rows2.at[0], semg)

    def chunk(i, carry):
        b = lax.rem(i, 2)
        nb = 1 - b
        nxt = i + 1

        @pl.when(nxt < NCHUNK)
        def _():
            pltpu.async_copy(row_hbm.at[wid].at[nxt], row2.at[nb], semi)
            pltpu.async_copy(col_hbm.at[wid].at[nxt], col2.at[nb], semi)
            pltpu.async_copy(wrep_hbm.at[wid].at[nxt], wrep2.at[nb], semi)

        pltpu.make_async_copy(xs_hbm.at[row2.at[b]], rows2.at[b], semg).wait()

        @pl.when(nxt < NCHUNK)
        def _():
            pltpu.make_async_copy(row_hbm.at[wid].at[nxt], row2.at[nb], semi).wait()
            pltpu.make_async_copy(col_hbm.at[wid].at[nxt], col2.at[nb], semi).wait()
            pltpu.make_async_copy(wrep_hbm.at[wid].at[nxt], wrep2.at[nb], semi).wait()
            pltpu.async_copy(xs_hbm.at[row2.at[nb]], rows2.at[nb], semg)

        def scale(e, carry2):
            wb = wrep2[b, e]
            for j in range(H // 16):
                rows2[b, e, pl.ds(j * 16, 16)] = rows2[b, e, pl.ds(j * 16, 16)] * wb
            return carry2
        lax.fori_loop(0, K, scale, 0)

        pltpu.sync_copy(rows2.at[b], acc_sh.at[col2.at[b]], add=True)
        return carry
    lax.fori_loop(0, NCHUNK, chunk, 0)

    plsc.subcore_barrier()
    for k in range(RPT // ZR):
        pltpu.sync_copy(acc_sh.at[pl.ds(s * RPT + k * ZR, ZR)],
                        out_hbm.at[c].at[pl.ds(s * RPT + k * ZR, ZR)])


_spmm_kernel = functools.partial(
    pl.kernel,
    out_type=jax.ShapeDtypeStruct((NC, NPAD, H), jnp.float32),
    mesh=_mesh,
    scratch_types=[
        pltpu.VMEM((2, K), jnp.int32),
        pltpu.VMEM((2, K), jnp.int32),
        pltpu.VMEM((2, K, 16), jnp.float32),
        pltpu.VMEM((2, K, H), jnp.float32),
        pltpu.VMEM_SHARED((NPAD, H), jnp.float32),
        pltpu.SemaphoreType.DMA,
        pltpu.SemaphoreType.DMA,
    ],
)(_spmm_body)


MB = 400
G = N // MB


def _tc1_body(degp_ref, x_ref, w1t_ref, dis_ref, xs1_ref):
    deg = degp_ref[0] + degp_ref[1] + 1.0
    dis = jnp.where(deg > 0, lax.rsqrt(deg), 0.0)
    dis_ref[...] = dis
    xw = jnp.dot(x_ref[...], w1t_ref[...], preferred_element_type=jnp.float32)
    xs1_ref[...] = xw * dis


_tc1 = pl.pallas_call(
    _tc1_body,
    grid=(G,),
    in_specs=[
        pl.BlockSpec((NC, MB, 1), lambda i: (0, i, 0)),
        pl.BlockSpec((MB, H), lambda i: (i, 0)),
        pl.BlockSpec((H, H), lambda i: (0, 0)),
    ],
    out_specs=[
        pl.BlockSpec((MB, 1), lambda i: (i, 0)),
        pl.BlockSpec((MB, H), lambda i: (i, 0)),
    ],
    out_shape=[
        jax.ShapeDtypeStruct((N, 1), jnp.float32),
        jax.ShapeDtypeStruct((N, H), jnp.float32),
    ],
)


def _tc2_body(accp_ref, xs1_ref, dis_ref, b1_ref, w2t_ref, xs2_ref):
    dis = dis_ref[...]
    pre = dis * (accp_ref[0] + accp_ref[1] + xs1_ref[...]) + b1_ref[...]
    h = jnp.maximum(pre, 0.0)
    xs2_ref[...] = jnp.dot(h, w2t_ref[...],
                           preferred_element_type=jnp.float32) * dis


_tc2 = pl.pallas_call(
    _tc2_body,
    grid=(G,),
    in_specs=[
        pl.BlockSpec((NC, MB, H), lambda i: (0, i, 0)),
        pl.BlockSpec((MB, H), lambda i: (i, 0)),
        pl.BlockSpec((MB, 1), lambda i: (i, 0)),
        pl.BlockSpec((1, H), lambda i: (0, 0)),
        pl.BlockSpec((H, H), lambda i: (0, 0)),
    ],
    out_specs=pl.BlockSpec((MB, H), lambda i: (i, 0)),
    out_shape=jax.ShapeDtypeStruct((N, H), jnp.float32),
)


def _tc3_body(accp_ref, xs2_ref, dis_ref, b2_ref, out_ref):
    dis = dis_ref[...]
    pre = dis * (accp_ref[0] + accp_ref[1] + xs2_ref[...]) + b2_ref[...]
    out_ref[...] = jnp.maximum(pre, 0.0)


_tc3 = pl.pallas_call(
    _tc3_body,
    grid=(G,),
    in_specs=[
        pl.BlockSpec((NC, MB, H), lambda i: (0, i, 0)),
        pl.BlockSpec((MB, H), lambda i: (i, 0)),
        pl.BlockSpec((MB, 1), lambda i: (i, 0)),
        pl.BlockSpec((1, H), lambda i: (0, 0)),
    ],
    out_specs=pl.BlockSpec((MB, H), lambda i: (i, 0)),
    out_shape=jax.ShapeDtypeStruct((N, H), jnp.float32),
)


def kernel(x, edge_index, edge_mask, W1, b1, W2, b2):
    row = edge_index[0]
    col = edge_index[1]
    w = edge_mask
    row3 = row.reshape(NW, NCHUNK, K)
    col3 = col.reshape(NW, NCHUNK, K)
    w3 = w.reshape(NW, NCHUNK, K)
    wrep = jnp.broadcast_to(w[:, None], (E, 16)).reshape(NW, NCHUNK, K, 16)

    degp = _deg_kernel(col3, w3)
    degp2 = degp.reshape(NC, NPAD, 1)
    dis, xs1 = _tc1(degp2, x, W1.T)
    acc1 = _spmm_kernel(xs1, row3, col3, wrep)
    xs2 = _tc2(acc1, xs1, dis, b1.reshape(1, H), W2.T)
    acc2 = _spmm_kernel(xs2, row3, col3, wrep)
    out = _tc3(acc2, xs2, dis, b2.reshape(1, H))
    return out


# speedup vs baseline: 7.0853x; 1.0005x over previous
import functools

import jax
import jax.numpy as jnp
from jax import lax
from jax.experimental import pallas as pl
from jax.experimental.pallas import tpu as pltpu
from jax.experimental.pallas import tpu_sc as plsc

N = 10000
E = 320000
H = 128

NC = 2
NS = 16
NW = NC * NS
EPT = E // NW
K = 80
NCHUNK = EPT // K
NPAD = 10240
RPT = NPAD // NS
ZR = 128
DSL = NPAD // NS

_mesh = plsc.VectorSubcoreMesh(
    core_axis_name="c", subcore_axis_name="s", num_cores=NC, num_subcores=NS)


def _deg_body(col_hbm, w_hbm, out_hbm, col_v, w_v, zbuf, deg_sh, sem):
    c = lax.axis_index("c")
    s = lax.axis_index("s")
    wid = c * NS + s
    zero16 = jnp.zeros((16,), jnp.float32)

    def zb(r, carry):
        zbuf[pl.ds(r * 16, 16)] = zero16
        return carry
    lax.fori_loop(0, DSL // 16, zb, 0)
    pltpu.sync_copy(zbuf, deg_sh.at[pl.ds(s * DSL, DSL)])

    pltpu.sync_copy(col_hbm.at[wid], col_v)
    pltpu.sync_copy(w_hbm.at[wid], w_v)

    plsc.subcore_barrier()

    def chunk(i, carry):
        pltpu.sync_copy(w_v.at[i], deg_sh.at[col_v.at[i]], add=True)
        return carry
    lax.fori_loop(0, NCHUNK, chunk, 0)

    plsc.subcore_barrier()
    pltpu.sync_copy(deg_sh.at[pl.ds(s * DSL, DSL)],
                    out_hbm.at[c].at[pl.ds(s * DSL, DSL)])


_deg_kernel = functools.partial(
    pl.kernel,
    out_type=jax.ShapeDtypeStruct((NC, NPAD), jnp.float32),
    mesh=_mesh,
    scratch_types=[
        pltpu.VMEM((NCHUNK, K), jnp.int32),
        pltpu.VMEM((NCHUNK, K), jnp.float32),
        pltpu.VMEM((DSL,), jnp.float32),
        pltpu.VMEM_SHARED((NPAD,), jnp.float32),
        pltpu.SemaphoreType.DMA,
    ],
)(_deg_body)


def _spmm_body(xs_hbm, row_hbm, col_hbm, wrep_hbm, out_hbm,
               row3, col3, wrep3, rows2, acc_sh, semi, semg, sems):
    c = lax.axis_index("c")
    s = lax.axis_index("s")
    wid = c * NS + s
    zero16 = jnp.zeros((16,), jnp.float32)

    def zb(r, carry):
        for j in range(H // 16):
            rows2[0, r, pl.ds(j * 16, 16)] = zero16
        return carry
    lax.fori_loop(0, K, zb, 0)
    for k in range(RPT // K):
        pltpu.sync_copy(rows2.at[0], acc_sh.at[pl.ds(s * RPT + k * K, K)])

    plsc.subcore_barrier()

    pltpu.sync_copy(row_hbm.at[wid].at[0], row3.at[0])
    pltpu.sync_copy(col_hbm.at[wid].at[0], col3.at[0])
    pltpu.sync_copy(wrep_hbm.at[wid].at[0], wrep3.at[0])
    pltpu.async_copy(xs_hbm.at[row3.at[0]], rows2.at[0], semg)

    def chunk(i, carry):
        b = lax.rem(i, 2)
        nb = 1 - b
        nxt = i + 1

        @pl.when(i >= 1)
        def _():
            pltpu.make_async_copy(rows2.at[nb],
                                  acc_sh.at[col3.at[nb]], sems).wait()

        @pl.when(nxt < NCHUNK)
        def _():
            pltpu.async_copy(row_hbm.at[wid].at[nxt], row3.at[nb], semi)
            pltpu.async_copy(col_hbm.at[wid].at[nxt], col3.at[nb], semi)
            pltpu.async_copy(wrep_hbm.at[wid].at[nxt], wrep3.at[nb], semi)

        pltpu.make_async_copy(xs_hbm.at[row3.at[b]], rows2.at[b], semg).wait()

        @pl.when(nxt < NCHUNK)
        def _():
            pltpu.make_async_copy(row_hbm.at[wid].at[nxt], row3.at[nb], semi).wait()
            pltpu.make_async_copy(col_hbm.at[wid].at[nxt], col3.at[nb], semi).wait()
            pltpu.make_async_copy(wrep_hbm.at[wid].at[nxt], wrep3.at[nb], semi).wait()
            pltpu.async_copy(xs_hbm.at[row3.at[nb]], rows2.at[nb], semg)

        def scale(e, carry2):
            wb = wrep3[b, e]
            for j in range(H // 16):
                rows2[b, e, pl.ds(j * 16, 16)] = rows2[b, e, pl.ds(j * 16, 16)] * wb
            return carry2
        lax.fori_loop(0, K, scale, 0)

        pltpu.async_copy(rows2.at[b], acc_sh.at[col3.at[b]], sems)
        return carry
    lax.fori_loop(0, NCHUNK, chunk, 0)

    pltpu.make_async_copy(rows2.at[(NCHUNK - 1) % 2],
                          acc_sh.at[col3.at[(NCHUNK - 1) % 2]], sems).wait()

    plsc.subcore_barrier()
    for k in range(RPT // ZR):
        pltpu.sync_copy(acc_sh.at[pl.ds(s * RPT + k * ZR, ZR)],
                        out_hbm.at[c].at[pl.ds(s * RPT + k * ZR, ZR)])


_spmm_kernel = functools.partial(
    pl.kernel,
    out_type=jax.ShapeDtypeStruct((NC, NPAD, H), jnp.float32),
    mesh=_mesh,
    scratch_types=[
        pltpu.VMEM((2, K), jnp.int32),
        pltpu.VMEM((2, K), jnp.int32),
        pltpu.VMEM((2, K, 16), jnp.float32),
        pltpu.VMEM((2, K, H), jnp.float32),
        pltpu.VMEM_SHARED((NPAD, H), jnp.float32),
        pltpu.SemaphoreType.DMA,
        pltpu.SemaphoreType.DMA,
        pltpu.SemaphoreType.DMA,
    ],
)(_spmm_body)


MB = 400
G = N // MB


def _tc1_body(degp_ref, x_ref, w1t_ref, dis_ref, xs1_ref):
    deg = degp_ref[0] + degp_ref[1] + 1.0
    dis = jnp.where(deg > 0, lax.rsqrt(deg), 0.0)
    dis_ref[...] = dis
    xw = jnp.dot(x_ref[...], w1t_ref[...], preferred_element_type=jnp.float32)
    xs1_ref[...] = xw * dis


_tc1 = pl.pallas_call(
    _tc1_body,
    grid=(G,),
    in_specs=[
        pl.BlockSpec((NC, MB, 1), lambda i: (0, i, 0)),
        pl.BlockSpec((MB, H), lambda i: (i, 0)),
        pl.BlockSpec((H, H), lambda i: (0, 0)),
    ],
    out_specs=[
        pl.BlockSpec((MB, 1), lambda i: (i, 0)),
        pl.BlockSpec((MB, H), lambda i: (i, 0)),
    ],
    out_shape=[
        jax.ShapeDtypeStruct((N, 1), jnp.float32),
        jax.ShapeDtypeStruct((N, H), jnp.float32),
    ],
)


def _tc2_body(accp_ref, xs1_ref, dis_ref, b1_ref, w2t_ref, xs2_ref):
    dis = dis_ref[...]
    pre = dis * (accp_ref[0] + accp_ref[1] + xs1_ref[...]) + b1_ref[...]
    h = jnp.maximum(pre, 0.0)
    xs2_ref[...] = jnp.dot(h, w2t_ref[...],
                           preferred_element_type=jnp.float32) * dis


_tc2 = pl.pallas_call(
    _tc2_body,
    grid=(G,),
    in_specs=[
        pl.BlockSpec((NC, MB, H), lambda i: (0, i, 0)),
        pl.BlockSpec((MB, H), lambda i: (i, 0)),
        pl.BlockSpec((MB, 1), lambda i: (i, 0)),
        pl.BlockSpec((1, H), lambda i: (0, 0)),
        pl.BlockSpec((H, H), lambda i: (0, 0)),
    ],
    out_specs=pl.BlockSpec((MB, H), lambda i: (i, 0)),
    out_shape=jax.ShapeDtypeStruct((N, H), jnp.float32),
)


def _tc3_body(accp_ref, xs2_ref, dis_ref, b2_ref, out_ref):
    dis = dis_ref[...]
    pre = dis * (accp_ref[0] + accp_ref[1] + xs2_ref[...]) + b2_ref[...]
    out_ref[...] = jnp.maximum(pre, 0.0)


_tc3 = pl.pallas_call(
    _tc3_body,
    grid=(G,),
    in_specs=[
        pl.BlockSpec((NC, MB, H), lambda i: (0, i, 0)),
        pl.BlockSpec((MB, H), lambda i: (i, 0)),
        pl.BlockSpec((MB, 1), lambda i: (i, 0)),
        pl.BlockSpec((1, H), lambda i: (0, 0)),
    ],
    out_specs=pl.BlockSpec((MB, H), lambda i: (i, 0)),
    out_shape=jax.ShapeDtypeStruct((N, H), jnp.float32),
)


def kernel(x, edge_index, edge_mask, W1, b1, W2, b2):
    row = edge_index[0]
    col = edge_index[1]
    w = edge_mask
    row3 = row.reshape(NW, NCHUNK, K)
    col3 = col.reshape(NW, NCHUNK, K)
    w3 = w.reshape(NW, NCHUNK, K)
    wrep = jnp.broadcast_to(w[:, None], (E, 16)).reshape(NW, NCHUNK, K, 16)

    degp = _deg_kernel(col3, w3)
    degp2 = degp.reshape(NC, NPAD, 1)
    dis, xs1 = _tc1(degp2, x, W1.T)
    acc1 = _spmm_kernel(xs1, row3, col3, wrep)
    xs2 = _tc2(acc1, xs1, dis, b1.reshape(1, H), W2.T)
    acc2 = _spmm_kernel(xs2, row3, col3, wrep)
    out = _tc3(acc2, xs2, dis, b2.reshape(1, H))
    return out


# speedup vs baseline: 8.9826x; 1.2678x over previous
import functools

import jax
import jax.numpy as jnp
from jax import lax
from jax.experimental import pallas as pl
from jax.experimental.pallas import tpu as pltpu
from jax.experimental.pallas import tpu_sc as plsc

N = 10000
E = 320000
H = 128

NC = 2
NS = 16
NW = NC * NS
EPT = E // NW
K = 80
NCHUNK = EPT // K
NPAD = 10240
RPT = NPAD // NS
ZR = 128
DSL = NPAD // NS

_mesh = plsc.VectorSubcoreMesh(
    core_axis_name="c", subcore_axis_name="s", num_cores=NC, num_subcores=NS)


def _deg_body(col_hbm, w_hbm, out_hbm, col_v, w_v, zbuf, deg_sh, sem):
    c = lax.axis_index("c")
    s = lax.axis_index("s")
    wid = c * NS + s
    zero16 = jnp.zeros((16,), jnp.float32)

    def zb(r, carry):
        zbuf[pl.ds(r * 16, 16)] = zero16
        return carry
    lax.fori_loop(0, DSL // 16, zb, 0)
    pltpu.sync_copy(zbuf, deg_sh.at[pl.ds(s * DSL, DSL)])

    pltpu.sync_copy(col_hbm.at[wid], col_v)
    pltpu.sync_copy(w_hbm.at[wid], w_v)

    plsc.subcore_barrier()

    def chunk(i, carry):
        pltpu.sync_copy(w_v.at[i], deg_sh.at[col_v.at[i]], add=True)
        return carry
    lax.fori_loop(0, NCHUNK, chunk, 0)

    plsc.subcore_barrier()
    pltpu.sync_copy(deg_sh.at[pl.ds(s * DSL, DSL)],
                    out_hbm.at[c].at[pl.ds(s * DSL, DSL)])


_deg_kernel = functools.partial(
    pl.kernel,
    out_type=jax.ShapeDtypeStruct((NC, NPAD), jnp.float32),
    mesh=_mesh,
    scratch_types=[
        pltpu.VMEM((NCHUNK, K), jnp.int32),
        pltpu.VMEM((NCHUNK, K), jnp.float32),
        pltpu.VMEM((DSL,), jnp.float32),
        pltpu.VMEM_SHARED((NPAD,), jnp.float32),
        pltpu.SemaphoreType.DMA,
    ],
)(_deg_body)


_BCAST_DN = lax.GatherDimensionNumbers(
    offset_dims=(), collapsed_slice_dims=(0,), start_index_map=(0,))


def _lane_bcast(vec, l):
    return lax.gather(vec, jnp.full((16, 1), l, jnp.int32), _BCAST_DN, (1,),
                      mode=lax.GatherScatterMode.PROMISE_IN_BOUNDS)


def _spmm_body(xs_hbm, rc_hbm, w_hbm, out_hbm,
               rc2, w_v, rows2, acc_sh, semi, semg):
    c = lax.axis_index("c")
    s = lax.axis_index("s")
    wid = c * NS + s
    zero16 = jnp.zeros((16,), jnp.float32)

    def zb(r, carry):
        for j in range(H // 16):
            rows2[0, r, pl.ds(j * 16, 16)] = zero16
        return carry
    lax.fori_loop(0, K, zb, 0)
    for k in range(RPT // K):
        pltpu.sync_copy(rows2.at[0], acc_sh.at[pl.ds(s * RPT + k * K, K)])

    plsc.subcore_barrier()

    pltpu.sync_copy(w_hbm.at[wid], w_v)
    pltpu.sync_copy(rc_hbm.at[wid].at[0], rc2.at[0])
    pltpu.async_copy(xs_hbm.at[rc2.at[0].at[0]], rows2.at[0], semg)

    def chunk(i, carry):
        b = lax.rem(i, 2)
        nb = 1 - b
        nxt = i + 1

        @pl.when(nxt < NCHUNK)
        def _():
            pltpu.async_copy(rc_hbm.at[wid].at[nxt], rc2.at[nb], semi)

        pltpu.make_async_copy(xs_hbm.at[rc2.at[b].at[0]],
                              rows2.at[b], semg).wait()

        @pl.when(nxt < NCHUNK)
        def _():
            pltpu.make_async_copy(rc_hbm.at[wid].at[nxt], rc2.at[nb], semi).wait()
            pltpu.async_copy(xs_hbm.at[rc2.at[nb].at[0]], rows2.at[nb], semg)

        def scale(g, carry2):
            wv16 = w_v[i, pl.ds(g * 16, 16)]
            for l in range(16):
                e = g * 16 + l
                wb = _lane_bcast(wv16, l)
                for j in range(H // 16):
                    rows2[b, e, pl.ds(j * 16, 16)] = (
                        rows2[b, e, pl.ds(j * 16, 16)] * wb)
            return carry2
        lax.fori_loop(0, K // 16, scale, 0)

        pltpu.sync_copy(rows2.at[b], acc_sh.at[rc2.at[b].at[1]], add=True)
        return carry
    lax.fori_loop(0, NCHUNK, chunk, 0)

    plsc.subcore_barrier()
    for k in range(RPT // ZR):
        pltpu.sync_copy(acc_sh.at[pl.ds(s * RPT + k * ZR, ZR)],
                        out_hbm.at[c].at[pl.ds(s * RPT + k * ZR, ZR)])


_spmm_kernel = functools.partial(
    pl.kernel,
    out_type=jax.ShapeDtypeStruct((NC, NPAD, H), jnp.float32),
    mesh=_mesh,
    scratch_types=[
        pltpu.VMEM((2, 2, K), jnp.int32),
        pltpu.VMEM((NCHUNK, K), jnp.float32),
        pltpu.VMEM((2, K, H), jnp.float32),
        pltpu.VMEM_SHARED((NPAD, H), jnp.float32),
        pltpu.SemaphoreType.DMA,
        pltpu.SemaphoreType.DMA,
    ],
)(_spmm_body)


MB = 400
G = N // MB


def _tc1_body(degp_ref, x_ref, w1t_ref, dis_ref, xs1_ref):
    deg = degp_ref[0] + degp_ref[1] + 1.0
    dis = jnp.where(deg > 0, lax.rsqrt(deg), 0.0)
    dis_ref[...] = dis
    xw = jnp.dot(x_ref[...], w1t_ref[...], preferred_element_type=jnp.float32)
    xs1_ref[...] = xw * dis


_tc1 = pl.pallas_call(
    _tc1_body,
    grid=(G,),
    in_specs=[
        pl.BlockSpec((NC, MB, 1), lambda i: (0, i, 0)),
        pl.BlockSpec((MB, H), lambda i: (i, 0)),
        pl.BlockSpec((H, H), lambda i: (0, 0)),
    ],
    out_specs=[
        pl.BlockSpec((MB, 1), lambda i: (i, 0)),
        pl.BlockSpec((MB, H), lambda i: (i, 0)),
    ],
    out_shape=[
        jax.ShapeDtypeStruct((N, 1), jnp.float32),
        jax.ShapeDtypeStruct((N, H), jnp.float32),
    ],
)


def _tc2_body(accp_ref, xs1_ref, dis_ref, b1_ref, w2t_ref, xs2_ref):
    dis = dis_ref[...]
    pre = dis * (accp_ref[0] + accp_ref[1] + xs1_ref[...]) + b1_ref[...]
    h = jnp.maximum(pre, 0.0)
    xs2_ref[...] = jnp.dot(h, w2t_ref[...],
                           preferred_element_type=jnp.float32) * dis


_tc2 = pl.pallas_call(
    _tc2_body,
    grid=(G,),
    in_specs=[
        pl.BlockSpec((NC, MB, H), lambda i: (0, i, 0)),
        pl.BlockSpec((MB, H), lambda i: (i, 0)),
        pl.BlockSpec((MB, 1), lambda i: (i, 0)),
        pl.BlockSpec((1, H), lambda i: (0, 0)),
        pl.BlockSpec((H, H), lambda i: (0, 0)),
    ],
    out_specs=pl.BlockSpec((MB, H), lambda i: (i, 0)),
    out_shape=jax.ShapeDtypeStruct((N, H), jnp.float32),
)


def _tc3_body(accp_ref, xs2_ref, dis_ref, b2_ref, out_ref):
    dis = dis_ref[...]
    pre = dis * (accp_ref[0] + accp_ref[1] + xs2_ref[...]) + b2_ref[...]
    out_ref[...] = jnp.maximum(pre, 0.0)


_tc3 = pl.pallas_call(
    _tc3_body,
    grid=(G,),
    in_specs=[
        pl.BlockSpec((NC, MB, H), lambda i: (0, i, 0)),
        pl.BlockSpec((MB, H), lambda i: (i, 0)),
        pl.BlockSpec((MB, 1), lambda i: (i, 0)),
        pl.BlockSpec((1, H), lambda i: (0, 0)),
    ],
    out_specs=pl.BlockSpec((MB, H), lambda i: (i, 0)),
    out_shape=jax.ShapeDtypeStruct((N, H), jnp.float32),
)


def kernel(x, edge_index, edge_mask, W1, b1, W2, b2):
    row = edge_index[0]
    col = edge_index[1]
    w = edge_mask
    row3 = row.reshape(NW, NCHUNK, K)
    col3 = col.reshape(NW, NCHUNK, K)
    w3 = w.reshape(NW, NCHUNK, K)
    rc = jnp.stack([row3, col3], axis=2)

    degp = _deg_kernel(col3, w3)
    degp2 = degp.reshape(NC, NPAD, 1)
    dis, xs1 = _tc1(degp2, x, W1.T)
    acc1 = _spmm_kernel(xs1, rc, w3)
    xs2 = _tc2(acc1, xs1, dis, b1.reshape(1, H), W2.T)
    acc2 = _spmm_kernel(xs2, rc, w3)
    out = _tc3(acc2, xs2, dis, b2.reshape(1, H))
    return out


# speedup vs baseline: 10.3293x; 1.1499x over previous
import functools

import jax
import jax.numpy as jnp
from jax import lax
from jax.experimental import pallas as pl
from jax.experimental.pallas import tpu as pltpu
from jax.experimental.pallas import tpu_sc as plsc

N = 10000
E = 320000
H = 128

NC = 2
NS = 16
NW = NC * NS
EPT = E // NW
K = 80
NCHUNK = EPT // K
WCH = 5
NPAD = 10240
RPT = NPAD // NS
ZR = 128
DSL = NPAD // NS

_mesh = plsc.VectorSubcoreMesh(
    core_axis_name="c", subcore_axis_name="s", num_cores=NC, num_subcores=NS)


def _deg_body(col_hbm, w_hbm, out_hbm, col_v, w_v, zbuf, deg_sh, sem):
    c = lax.axis_index("c")
    s = lax.axis_index("s")
    wid = c * NS + s
    zero16 = jnp.zeros((16,), jnp.float32)

    def zb(r, carry):
        zbuf[pl.ds(r * 16, 16)] = zero16
        return carry
    lax.fori_loop(0, DSL // 16, zb, 0)
    pltpu.sync_copy(zbuf, deg_sh.at[pl.ds(s * DSL, DSL)])

    pltpu.sync_copy(col_hbm.at[wid], col_v)
    pltpu.sync_copy(w_hbm.at[wid], w_v)

    plsc.subcore_barrier()

    def chunk(i, carry):
        pltpu.sync_copy(w_v.at[i], deg_sh.at[col_v.at[i]], add=True)
        return carry
    lax.fori_loop(0, NCHUNK, chunk, 0)

    plsc.subcore_barrier()
    pltpu.sync_copy(deg_sh.at[pl.ds(s * DSL, DSL)],
                    out_hbm.at[c].at[pl.ds(s * DSL, DSL)])


_deg_kernel = functools.partial(
    pl.kernel,
    out_type=jax.ShapeDtypeStruct((NC, NPAD), jnp.float32),
    mesh=_mesh,
    scratch_types=[
        pltpu.VMEM((NCHUNK, K), jnp.int32),
        pltpu.VMEM((NCHUNK, K), jnp.float32),
        pltpu.VMEM((DSL,), jnp.float32),
        pltpu.VMEM_SHARED((NPAD,), jnp.float32),
        pltpu.SemaphoreType.DMA,
    ],
)(_deg_body)


_BCAST_DN = lax.GatherDimensionNumbers(
    offset_dims=(), collapsed_slice_dims=(0,), start_index_map=(0,))


def _lane_bcast(vec, l):
    return lax.gather(vec, jnp.full((16, 1), l, jnp.int32), _BCAST_DN, (1,),
                      mode=lax.GatherScatterMode.PROMISE_IN_BOUNDS)


def _spmm_body(xs_hbm, rc_hbm, w_hbm, out_hbm,
               rcb, wvb, rows2, acc_sh, semi, semg):
    c = lax.axis_index("c")
    s = lax.axis_index("s")
    wid = c * NS + s
    zero16 = jnp.zeros((16,), jnp.float32)

    def zb(r, carry):
        for j in range(H // 16):
            rows2[0, r, pl.ds(j * 16, 16)] = zero16
        return carry
    lax.fori_loop(0, K, zb, 0)
    for k in range(RPT // K):
        pltpu.sync_copy(rows2.at[0], acc_sh.at[pl.ds(s * RPT + k * K, K)])

    plsc.subcore_barrier()

    pltpu.sync_copy(rc_hbm.at[wid].at[0], rcb.at[0])
    pltpu.sync_copy(w_hbm.at[wid].at[0], wvb.at[0])
    pltpu.async_copy(xs_hbm.at[rcb.at[0].at[0].at[0]], rows2.at[0], semg)
    pltpu.async_copy(rc_hbm.at[wid].at[1], rcb.at[1], semi)
    pltpu.async_copy(w_hbm.at[wid].at[1], wvb.at[1], semi)

    def chunk(i, carry):
        b = lax.rem(i, 2)
        nb = 1 - b
        q = lax.rem(i, WCH)
        wslot = lax.rem(i // WCH, 3)
        nxt = i + 1
        nq = lax.rem(nxt, WCH)
        nwslot = lax.rem(nxt // WCH, 3)

        pltpu.make_async_copy(xs_hbm.at[rcb.at[wslot].at[q].at[0]],
                              rows2.at[b], semg).wait()

        @pl.when(jnp.logical_and(nxt < NCHUNK, nq == 0))
        def _():
            pltpu.make_async_copy(rc_hbm.at[wid].at[nxt // WCH],
                                  rcb.at[nwslot], semi).wait()
            pltpu.make_async_copy(w_hbm.at[wid].at[nxt // WCH],
                                  wvb.at[nwslot], semi).wait()

        @pl.when(jnp.logical_and(nq == 0, nxt + WCH < NCHUNK))
        def _():
            pltpu.async_copy(rc_hbm.at[wid].at[nxt // WCH + 1],
                             rcb.at[lax.rem(nxt // WCH + 1, 3)], semi)
            pltpu.async_copy(w_hbm.at[wid].at[nxt // WCH + 1],
                             wvb.at[lax.rem(nxt // WCH + 1, 3)], semi)

        @pl.when(nxt < NCHUNK)
        def _():
            pltpu.async_copy(xs_hbm.at[rcb.at[nwslot].at[nq].at[0]],
                             rows2.at[nb], semg)

        def scale(g, carry2):
            wv16 = wvb[wslot, q, pl.ds(g * 16, 16)]
            for l in range(16):
                e = g * 16 + l
                wb = _lane_bcast(wv16, l)
                for j in range(H // 16):
                    rows2[b, e, pl.ds(j * 16, 16)] = (
                        rows2[b, e, pl.ds(j * 16, 16)] * wb)
            return carry2
        lax.fori_loop(0, K // 16, scale, 0)

        pltpu.sync_copy(rows2.at[b], acc_sh.at[rcb.at[wslot].at[q].at[1]],
                        add=True)
        return carry
    lax.fori_loop(0, NCHUNK, chunk, 0)

    plsc.subcore_barrier()
    for k in range(RPT // ZR):
        pltpu.sync_copy(acc_sh.at[pl.ds(s * RPT + k * ZR, ZR)],
                        out_hbm.at[c].at[pl.ds(s * RPT + k * ZR, ZR)])


_spmm_kernel = functools.partial(
    pl.kernel,
    out_type=jax.ShapeDtypeStruct((NC, NPAD, H), jnp.float32),
    mesh=_mesh,
    scratch_types=[
        pltpu.VMEM((3, WCH, 2, K), jnp.int32),
        pltpu.VMEM((3, WCH, K), jnp.float32),
        pltpu.VMEM((2, K, H), jnp.float32),
        pltpu.VMEM_SHARED((NPAD, H), jnp.float32),
        pltpu.SemaphoreType.DMA,
        pltpu.SemaphoreType.DMA,
    ],
)(_spmm_body)


MB = 2000
G = N // MB


def _tc1_body(degp_ref, x_ref, w1t_ref, dis_ref, xs1_ref):
    deg = degp_ref[0] + degp_ref[1] + 1.0
    dis = jnp.where(deg > 0, lax.rsqrt(deg), 0.0)
    dis_ref[...] = dis
    xw = jnp.dot(x_ref[...], w1t_ref[...], preferred_element_type=jnp.float32)
    xs1_ref[...] = xw * dis


_tc1 = pl.pallas_call(
    _tc1_body,
    grid=(G,),
    in_specs=[
        pl.BlockSpec((NC, MB, 1), lambda i: (0, i, 0)),
        pl.BlockSpec((MB, H), lambda i: (i, 0)),
        pl.BlockSpec((H, H), lambda i: (0, 0)),
    ],
    out_specs=[
        pl.BlockSpec((MB, 1), lambda i: (i, 0)),
        pl.BlockSpec((MB, H), lambda i: (i, 0)),
    ],
    out_shape=[
        jax.ShapeDtypeStruct((N, 1), jnp.float32),
        jax.ShapeDtypeStruct((N, H), jnp.float32),
    ],
)


def _tc2_body(accp_ref, xs1_ref, dis_ref, b1_ref, w2t_ref, xs2_ref):
    dis = dis_ref[...]
    pre = dis * (accp_ref[0] + accp_ref[1] + xs1_ref[...]) + b1_ref[...]
    h = jnp.maximum(pre, 0.0)
    xs2_ref[...] = jnp.dot(h, w2t_ref[...],
                           preferred_element_type=jnp.float32) * dis


_tc2 = pl.pallas_call(
    _tc2_body,
    grid=(G,),
    in_specs=[
        pl.BlockSpec((NC, MB, H), lambda i: (0, i, 0)),
        pl.BlockSpec((MB, H), lambda i: (i, 0)),
        pl.BlockSpec((MB, 1), lambda i: (i, 0)),
        pl.BlockSpec((1, H), lambda i: (0, 0)),
        pl.BlockSpec((H, H), lambda i: (0, 0)),
    ],
    out_specs=pl.BlockSpec((MB, H), lambda i: (i, 0)),
    out_shape=jax.ShapeDtypeStruct((N, H), jnp.float32),
)


def _tc3_body(accp_ref, xs2_ref, dis_ref, b2_ref, out_ref):
    dis = dis_ref[...]
    pre = dis * (accp_ref[0] + accp_ref[1] + xs2_ref[...]) + b2_ref[...]
    out_ref[...] = jnp.maximum(pre, 0.0)


_tc3 = pl.pallas_call(
    _tc3_body,
    grid=(G,),
    in_specs=[
        pl.BlockSpec((NC, MB, H), lambda i: (0, i, 0)),
        pl.BlockSpec((MB, H), lambda i: (i, 0)),
        pl.BlockSpec((MB, 1), lambda i: (i, 0)),
        pl.BlockSpec((1, H), lambda i: (0, 0)),
    ],
    out_specs=pl.BlockSpec((MB, H), lambda i: (i, 0)),
    out_shape=jax.ShapeDtypeStruct((N, H), jnp.float32),
)


def kernel(x, edge_index, edge_mask, W1, b1, W2, b2):
    row = edge_index[0]
    col = edge_index[1]
    w = edge_mask
    row3 = row.reshape(NW, NCHUNK, K)
    col3 = col.reshape(NW, NCHUNK, K)
    w3 = w.reshape(NW, NCHUNK, K)
    rc = jnp.stack([row3, col3], axis=2).reshape(
        NW, NCHUNK // WCH, WCH, 2, K)
    w5 = w3.reshape(NW, NCHUNK // WCH, WCH, K)

    degp = _deg_kernel(col3, w3)
    degp2 = degp.reshape(NC, NPAD, 1)
    dis, xs1 = _tc1(degp2, x, W1.T)
    acc1 = _spmm_kernel(xs1, rc, w5)
    xs2 = _tc2(acc1, xs1, dis, b1.reshape(1, H), W2.T)
    acc2 = _spmm_kernel(xs2, rc, w5)
    out = _tc3(acc2, xs2, dis, b2.reshape(1, H))
    return out
